# baseline (speedup 1.0000x reference)
import functools

import jax
import jax.numpy as jnp
from jax.experimental import pallas as pl
from jax.experimental.pallas import tpu as pltpu


def _mm_kernel(a_ref, w_ref, m_ref, o_ref, acc_ref, *, k_steps, out_dtype):
    k = pl.program_id(2)

    @pl.when(k == 0)
    def _():
        acc_ref[...] = jnp.zeros_like(acc_ref)

    a = a_ref[...].astype(jnp.bfloat16)
    s = (w_ref[...] * m_ref[...]).astype(jnp.bfloat16)
    acc_ref[...] += jax.lax.dot_general(
        a, s, (((1,), (1,)), ((), ())), preferred_element_type=jnp.float32)

    @pl.when(k == k_steps - 1)
    def _():
        o_ref[...] = acc_ref[...].astype(out_dtype)


def _mm_bias_kernel(a_ref, w_ref, m_ref, b_ref, o_ref, acc_ref, *, k_steps,
                    out_dtype):
    k = pl.program_id(2)

    @pl.when(k == 0)
    def _():
        acc_ref[...] = jnp.zeros_like(acc_ref)

    a = a_ref[...].astype(jnp.bfloat16)
    s = (w_ref[...] * m_ref[...]).astype(jnp.bfloat16)
    acc_ref[...] += jax.lax.dot_general(
        a, s, (((1,), (1,)), ((), ())), preferred_element_type=jnp.float32)

    @pl.when(k == k_steps - 1)
    def _():
        o_ref[...] = (acc_ref[...] + b_ref[...]).astype(out_dtype)


def _masked_mm(a, w, mask, bias, out_dtype, bm, bn, bk):
    m_dim, k_dim = a.shape
    n_dim = w.shape[0]
    grid = (m_dim // bm, n_dim // bn, k_dim // bk)
    in_specs = [
        pl.BlockSpec((bm, bk), lambda m, n, k: (m, k)),
        pl.BlockSpec((bn, bk), lambda m, n, k: (n, k)),
        pl.BlockSpec((bn, bk), lambda m, n, k: (n, k)),
    ]
    args = (a, w, mask)
    if bias is None:
        body = functools.partial(_mm_kernel, k_steps=grid[2],
                                 out_dtype=out_dtype)
    else:
        body = functools.partial(_mm_bias_kernel, k_steps=grid[2],
                                 out_dtype=out_dtype)
        in_specs.append(pl.BlockSpec((1, bn), lambda m, n, k: (0, n)))
        args = args + (bias.reshape(1, -1),)
    return pl.pallas_call(
        body,
        grid=grid,
        in_specs=in_specs,
        out_specs=pl.BlockSpec((bm, bn), lambda m, n, k: (m, n)),
        out_shape=jax.ShapeDtypeStruct((m_dim, n_dim), out_dtype),
        scratch_shapes=[pltpu.VMEM((bm, bn), jnp.float32)],
        compiler_params=pltpu.CompilerParams(
            dimension_semantics=("arbitrary", "arbitrary", "arbitrary")),
    )(*args)


def kernel(U, W0, W1, W2, M0, M1, M2, bias):
    a1 = _masked_mm(U, W2, M2, None, jnp.bfloat16, 1024, 1024, 512)
    a2 = _masked_mm(a1, W1, M1, None, jnp.bfloat16, 1024, 1024, 512)
    out = _masked_mm(a2, W0, M0, bias, jnp.float32, 1024, 1024, 512)
    return out


# speedup vs baseline: 1.4526x; 1.4526x over previous
import functools

import jax
import jax.numpy as jnp
from jax.experimental import pallas as pl
from jax.experimental.pallas import tpu as pltpu


def _mm_kernel(a_ref, w_ref, m_ref, o_ref, acc_ref, *, k_steps, out_dtype):
    k = pl.program_id(2)

    @pl.when(k == 0)
    def _():
        acc_ref[...] = jnp.zeros_like(acc_ref)

    a = a_ref[...].astype(jnp.bfloat16)
    s = (w_ref[...] * m_ref[...]).astype(jnp.bfloat16)
    acc_ref[...] += jax.lax.dot_general(
        a, s, (((1,), (1,)), ((), ())), preferred_element_type=jnp.float32)

    @pl.when(k == k_steps - 1)
    def _():
        o_ref[...] = acc_ref[...].astype(out_dtype)


def _mm_bias_kernel(a_ref, w_ref, m_ref, b_ref, o_ref, acc_ref, *, k_steps,
                    out_dtype):
    k = pl.program_id(2)

    @pl.when(k == 0)
    def _():
        acc_ref[...] = jnp.zeros_like(acc_ref)

    a = a_ref[...].astype(jnp.bfloat16)
    s = (w_ref[...] * m_ref[...]).astype(jnp.bfloat16)
    acc_ref[...] += jax.lax.dot_general(
        a, s, (((1,), (1,)), ((), ())), preferred_element_type=jnp.float32)

    @pl.when(k == k_steps - 1)
    def _():
        o_ref[...] = (acc_ref[...] + b_ref[...]).astype(out_dtype)


def _masked_mm(a, w, mask, bias, out_dtype, bm, bn, bk):
    m_dim, k_dim = a.shape
    n_dim = w.shape[0]
    grid = (m_dim // bm, n_dim // bn, k_dim // bk)
    in_specs = [
        pl.BlockSpec((bm, bk), lambda m, n, k: (m, k)),
        pl.BlockSpec((bn, bk), lambda m, n, k: (n, k)),
        pl.BlockSpec((bn, bk), lambda m, n, k: (n, k)),
    ]
    args = (a, w, mask)
    if bias is None:
        body = functools.partial(_mm_kernel, k_steps=grid[2],
                                 out_dtype=out_dtype)
    else:
        body = functools.partial(_mm_bias_kernel, k_steps=grid[2],
                                 out_dtype=out_dtype)
        in_specs.append(pl.BlockSpec((1, bn), lambda m, n, k: (0, n)))
        args = args + (bias.reshape(1, -1),)
    return pl.pallas_call(
        body,
        grid=grid,
        in_specs=in_specs,
        out_specs=pl.BlockSpec((bm, bn), lambda m, n, k: (m, n)),
        out_shape=jax.ShapeDtypeStruct((m_dim, n_dim), out_dtype),
        scratch_shapes=[pltpu.VMEM((bm, bn), jnp.float32)],
        compiler_params=pltpu.CompilerParams(
            dimension_semantics=("arbitrary", "arbitrary", "arbitrary")),
    )(*args)


def kernel(U, W0, W1, W2, M0, M1, M2, bias):
    a1 = _masked_mm(U, W2, M2, None, jnp.bfloat16, 2048, 1024, 1024)
    a2 = _masked_mm(a1, W1, M1, None, jnp.bfloat16, 2048, 1024, 1024)
    out = _masked_mm(a2, W0, M0, bias, jnp.float32, 2048, 1024, 1024)
    return out


# speedup vs baseline: 1.4638x; 1.0077x over previous
import functools

import jax
import jax.numpy as jnp
from jax.experimental import pallas as pl
from jax.experimental.pallas import tpu as pltpu


def _mm_kernel(a_ref, w_ref, m_ref, o_ref, acc_ref, *, k_steps, out_dtype):
    k = pl.program_id(2)

    @pl.when(k == 0)
    def _():
        acc_ref[...] = jnp.zeros_like(acc_ref)

    a = a_ref[...].astype(jnp.bfloat16)
    s = (w_ref[...] * m_ref[...]).astype(jnp.bfloat16)
    acc_ref[...] += jax.lax.dot_general(
        a, s, (((1,), (1,)), ((), ())), preferred_element_type=jnp.float32)

    @pl.when(k == k_steps - 1)
    def _():
        o_ref[...] = acc_ref[...].astype(out_dtype)


def _mm_bias_kernel(a_ref, w_ref, m_ref, b_ref, o_ref, acc_ref, *, k_steps,
                    out_dtype):
    k = pl.program_id(2)

    @pl.when(k == 0)
    def _():
        acc_ref[...] = jnp.zeros_like(acc_ref)

    a = a_ref[...].astype(jnp.bfloat16)
    s = (w_ref[...] * m_ref[...]).astype(jnp.bfloat16)
    acc_ref[...] += jax.lax.dot_general(
        a, s, (((1,), (1,)), ((), ())), preferred_element_type=jnp.float32)

    @pl.when(k == k_steps - 1)
    def _():
        o_ref[...] = (acc_ref[...] + b_ref[...]).astype(out_dtype)


def _masked_mm(a, w, mask, bias, out_dtype, bm, bn, bk):
    m_dim, k_dim = a.shape
    n_dim = w.shape[0]
    grid = (m_dim // bm, n_dim // bn, k_dim // bk)
    in_specs = [
        pl.BlockSpec((bm, bk), lambda m, n, k: (m, k)),
        pl.BlockSpec((bn, bk), lambda m, n, k: (n, k)),
        pl.BlockSpec((bn, bk), lambda m, n, k: (n, k)),
    ]
    args = (a, w, mask)
    if bias is None:
        body = functools.partial(_mm_kernel, k_steps=grid[2],
                                 out_dtype=out_dtype)
    else:
        body = functools.partial(_mm_bias_kernel, k_steps=grid[2],
                                 out_dtype=out_dtype)
        in_specs.append(pl.BlockSpec((1, bn), lambda m, n, k: (0, n)))
        args = args + (bias.reshape(1, -1),)
    return pl.pallas_call(
        body,
        grid=grid,
        in_specs=in_specs,
        out_specs=pl.BlockSpec((bm, bn), lambda m, n, k: (m, n)),
        out_shape=jax.ShapeDtypeStruct((m_dim, n_dim), out_dtype),
        scratch_shapes=[pltpu.VMEM((bm, bn), jnp.float32)],
        compiler_params=pltpu.CompilerParams(
            dimension_semantics=("arbitrary", "arbitrary", "arbitrary")),
    )(*args)


def kernel(U, W0, W1, W2, M0, M1, M2, bias):
    a1 = _masked_mm(U, W2, M2, None, jnp.bfloat16, 2048, 2048, 256)
    a2 = _masked_mm(a1, W1, M1, None, jnp.bfloat16, 2048, 1024, 1024)
    out = _masked_mm(a2, W0, M0, bias, jnp.float32, 2048, 1024, 1024)
    return out


# speedup vs baseline: 1.5028x; 1.0266x over previous
import functools

import jax
import jax.numpy as jnp
from jax.experimental import pallas as pl
from jax.experimental.pallas import tpu as pltpu


def _mm_kernel(a_ref, w_ref, m_ref, o_ref, acc_ref, *, k_steps, out_dtype):
    k = pl.program_id(2)

    def _dot():
        a = a_ref[...].astype(jnp.bfloat16)
        s = (w_ref[...] * m_ref[...]).astype(jnp.bfloat16)
        return jax.lax.dot_general(
            a, s, (((1,), (1,)), ((), ())), preferred_element_type=jnp.float32)

    if k_steps == 1:
        o_ref[...] = _dot().astype(out_dtype)
        return

    @pl.when(k == 0)
    def _():
        acc_ref[...] = _dot()

    @pl.when((k > 0) & (k < k_steps - 1))
    def _():
        acc_ref[...] += _dot()

    @pl.when(k == k_steps - 1)
    def _():
        o_ref[...] = (acc_ref[...] + _dot()).astype(out_dtype)


def _mm_bias_kernel(a_ref, w_ref, m_ref, b_ref, o_ref, acc_ref, *, k_steps,
                    out_dtype):
    k = pl.program_id(2)

    def _dot():
        a = a_ref[...].astype(jnp.bfloat16)
        s = (w_ref[...] * m_ref[...]).astype(jnp.bfloat16)
        return jax.lax.dot_general(
            a, s, (((1,), (1,)), ((), ())), preferred_element_type=jnp.float32)

    if k_steps == 1:
        o_ref[...] = (_dot() + b_ref[...]).astype(out_dtype)
        return

    @pl.when(k == 0)
    def _():
        acc_ref[...] = _dot()

    @pl.when((k > 0) & (k < k_steps - 1))
    def _():
        acc_ref[...] += _dot()

    @pl.when(k == k_steps - 1)
    def _():
        o_ref[...] = (acc_ref[...] + _dot() + b_ref[...]).astype(out_dtype)


def _masked_mm(a, w, mask, bias, out_dtype, bm, bn, bk):
    m_dim, k_dim = a.shape
    n_dim = w.shape[0]
    grid = (m_dim // bm, n_dim // bn, k_dim // bk)
    in_specs = [
        pl.BlockSpec((bm, bk), lambda m, n, k: (m, k)),
        pl.BlockSpec((bn, bk), lambda m, n, k: (n, k)),
        pl.BlockSpec((bn, bk), lambda m, n, k: (n, k)),
    ]
    args = (a, w, mask)
    if bias is None:
        body = functools.partial(_mm_kernel, k_steps=grid[2],
                                 out_dtype=out_dtype)
    else:
        body = functools.partial(_mm_bias_kernel, k_steps=grid[2],
                                 out_dtype=out_dtype)
        in_specs.append(pl.BlockSpec((1, bn), lambda m, n, k: (0, n)))
        args = args + (bias.reshape(1, -1),)
    return pl.pallas_call(
        body,
        grid=grid,
        in_specs=in_specs,
        out_specs=pl.BlockSpec((bm, bn), lambda m, n, k: (m, n)),
        out_shape=jax.ShapeDtypeStruct((m_dim, n_dim), out_dtype),
        scratch_shapes=[pltpu.VMEM((bm, bn), jnp.float32)],
        compiler_params=pltpu.CompilerParams(
            dimension_semantics=("arbitrary", "arbitrary", "arbitrary")),
    )(*args)


def kernel(U, W0, W1, W2, M0, M1, M2, bias):
    a1 = _masked_mm(U, W2, M2, None, jnp.bfloat16, 2048, 1024, 1024)
    a2 = _masked_mm(a1, W1, M1, None, jnp.bfloat16, 2048, 1024, 1024)
    out = _masked_mm(a2, W0, M0, bias, jnp.float32, 2048, 1024, 1024)
    return out
